# baseline (speedup 1.0000x reference)
import functools

import jax
import jax.numpy as jnp
from jax import lax
from jax.experimental import pallas as pl
from jax.experimental.pallas import tpu as pltpu
from jax.experimental.pallas import tpu_sc as plsc

VOCAB = 100000
EMBED = 768
BATCH = 4
SEQ = 2048

_INFO = plsc.get_sparse_core_info()
NC = _INFO.num_cores
NS = _INFO.num_subcores
NW = NC * NS
NFLAT = BATCH * SEQ
TPW = SEQ // NW
TCH = 8
NCHUNK = TPW // TCH
CROWS = BATCH * TCH
NLANE = 16
NVEC = EMBED // NLANE


def _make_kernel():
    mesh = plsc.VectorSubcoreMesh(core_axis_name="c", subcore_axis_name="s")

    @functools.partial(
        pl.kernel,
        mesh=mesh,
        out_type=jax.ShapeDtypeStruct((NFLAT, EMBED), jnp.float32),
        scratch_types=[
            pltpu.VMEM((BATCH, TPW), jnp.int32),
            pltpu.VMEM((TPW, EMBED), jnp.float32),
            pltpu.VMEM((3 * CROWS, EMBED), jnp.float32),
            pltpu.SemaphoreType.DMA((8,)),
        ],
    )
    def k(tok_hbm, idx_hbm, pos_hbm, out_hbm, idx_v, pos_res,
          tok_ring, sems):
        tok_bufs = tuple(tok_ring.at[pl.ds(i * CROWS, CROWS)] for i in range(3))
        sidx = sems.at[0]
        spos = sems.at[1]
        stoks = tuple(sems.at[2 + i] for i in range(3))
        souts = tuple(sems.at[5 + i] for i in range(3))

        wid = lax.axis_index("s") * NC + lax.axis_index("c")
        t0 = wid * TPW
        idx_dmas = [
            pltpu.async_copy(idx_hbm.at[b, pl.ds(t0, TPW)], idx_v.at[b], sidx)
            for b in range(BATCH)
        ]
        pos_dma = pltpu.async_copy(pos_hbm.at[pl.ds(t0, TPW)], pos_res, spos)
        for d in idx_dmas:
            d.wait()

        def start_gather(c):
            return [
                pltpu.async_copy(
                    tok_hbm.at[idx_v.at[b, pl.ds(c * TCH, TCH)]],
                    tok_bufs[c % 3].at[pl.ds(b * TCH, TCH)],
                    stoks[c % 3],
                )
                for b in range(BATCH)
            ]

        gathers = {0: start_gather(0)}
        pos_dma.wait()
        outs = {}
        for c in range(NCHUNK):
            if c - 2 in outs:
                for d in outs.pop(c - 2):
                    d.wait()
            if c + 1 < NCHUNK:
                gathers[c + 1] = start_gather(c + 1)
            for d in gathers.pop(c):
                d.wait()
            tok_v = tok_bufs[c % 3]

            outs[c] = []
            for b0 in (0, 2):
                @plsc.parallel_loop(0, TCH)
                def add_pos(t, _tok=tok_v, _c=c, _b0=b0):
                    for j0 in range(0, NVEC, 8):
                        pvs = [pos_res[_c * TCH + t,
                                       pl.ds((j0 + u) * NLANE, NLANE)]
                               for u in range(8)]
                        for u in range(8):
                            sl = pl.ds((j0 + u) * NLANE, NLANE)
                            for b in (_b0, _b0 + 1):
                                plsc.addupdate(_tok.at[b * TCH + t, sl], pvs[u])

                outs[c].extend(
                    pltpu.async_copy(
                        tok_v.at[pl.ds(b * TCH, TCH)],
                        out_hbm.at[pl.ds(b * SEQ + t0 + c * TCH, TCH)],
                        souts[c % 3],
                    )
                    for b in (b0, b0 + 1)
                )
        for c in sorted(outs):
            for d in outs.pop(c):
                d.wait()

    return k


_k = _make_kernel()


def kernel(values, tok_table, pos_table):
    out = _k(tok_table, values.astype(jnp.int32), pos_table)
    return out.reshape(BATCH, SEQ, EMBED)


# speedup vs baseline: 1.0205x; 1.0205x over previous
import functools

import jax
import jax.numpy as jnp
from jax import lax
from jax.experimental import pallas as pl
from jax.experimental.pallas import tpu as pltpu
from jax.experimental.pallas import tpu_sc as plsc

VOCAB = 100000
EMBED = 768
BATCH = 4
SEQ = 2048

_INFO = plsc.get_sparse_core_info()
NC = _INFO.num_cores
NS = _INFO.num_subcores
NW = NC * NS
NFLAT = BATCH * SEQ
TPW = SEQ // NW
TCH = 8
NCHUNK = TPW // TCH
CROWS = BATCH * TCH
NLANE = 16
NVEC = EMBED // NLANE


def _make_kernel():
    mesh = plsc.VectorSubcoreMesh(core_axis_name="c", subcore_axis_name="s")

    @functools.partial(
        pl.kernel,
        mesh=mesh,
        out_type=jax.ShapeDtypeStruct((NFLAT, EMBED), jnp.float32),
        scratch_types=[
            pltpu.VMEM((BATCH, TPW), jnp.int32),
            pltpu.VMEM((TPW, EMBED), jnp.float32),
            pltpu.VMEM((3 * CROWS, EMBED), jnp.float32),
            pltpu.SemaphoreType.DMA((8,)),
        ],
    )
    def k(tok_hbm, idx_hbm, pos_hbm, out_hbm, idx_v, pos_res,
          tok_ring, sems):
        tok_bufs = tuple(tok_ring.at[pl.ds(i * CROWS, CROWS)] for i in range(3))
        sidx = sems.at[0]
        spos = sems.at[1]
        stoks = tuple(sems.at[2 + i] for i in range(3))
        souts = tuple(sems.at[5 + i] for i in range(3))

        wid = lax.axis_index("s") * NC + lax.axis_index("c")
        t0 = wid * TPW
        idx_dmas = [
            pltpu.async_copy(idx_hbm.at[b, pl.ds(t0, TPW)], idx_v.at[b], sidx)
            for b in range(BATCH)
        ]
        pos_dma = pltpu.async_copy(pos_hbm.at[pl.ds(t0, TPW)], pos_res, spos)
        for d in idx_dmas:
            d.wait()

        def start_gather(c):
            return [
                pltpu.async_copy(
                    tok_hbm.at[idx_v.at[b, pl.ds(c * TCH, TCH)]],
                    tok_bufs[c % 3].at[pl.ds(b * TCH, TCH)],
                    stoks[c % 3],
                )
                for b in range(BATCH)
            ]

        gathers = {0: start_gather(0)}
        pos_dma.wait()
        outs = {}
        for c in range(NCHUNK):
            if c - 2 in outs:
                for d in outs.pop(c - 2):
                    d.wait()
            if c + 1 < NCHUNK:
                gathers[c + 1] = start_gather(c + 1)
            for d in gathers.pop(c):
                d.wait()
            tok_v = tok_bufs[c % 3]

            @plsc.parallel_loop(0, TCH)
            def add_pos(t, _tok=tok_v, _c=c):
                for j0 in range(0, NVEC, 8):
                    pvs = [pos_res[_c * TCH + t, pl.ds((j0 + u) * NLANE, NLANE)]
                           for u in range(8)]
                    for u in range(8):
                        sl = pl.ds((j0 + u) * NLANE, NLANE)
                        for b in range(BATCH):
                            plsc.addupdate(_tok.at[b * TCH + t, sl], pvs[u])

            outs[c] = [
                pltpu.async_copy(
                    tok_v.at[pl.ds(b * TCH, TCH)],
                    out_hbm.at[pl.ds(b * SEQ + t0 + c * TCH, TCH)],
                    souts[c % 3],
                )
                for b in range(BATCH)
            ]
        for c in sorted(outs):
            for d in outs.pop(c):
                d.wait()

    return k


_k = _make_kernel()


def kernel(values, tok_table, pos_table):
    out = _k(tok_table, values.astype(jnp.int32), pos_table)
    return out.reshape(BATCH, SEQ, EMBED)
